# baseline (speedup 1.0000x reference)
import functools

import jax
import jax.numpy as jnp
from jax import lax
from jax.experimental import pallas as pl
from jax.experimental.pallas import tpu as pltpu
from jax.experimental.pallas import tpu_sc as plsc

_L = 200
_D = 64
_B = 4096
_DT = _D // 8
_BT = _B // 128

_NUM_CORES = 2
_NUM_SUBCORES = 16

_L_SC = 104
_L_HALF = _L_SC // _NUM_CORES
_L_TC = _L - _L_SC
_LB = 8

def _expand_body(w_ref, o_ref):
    for dt in range(_DT):
        block = w_ref[:, dt * 8:(dt + 1) * 8]
        o_ref[dt] = jnp.broadcast_to(block[:, :, None], (_L, 8, 128))


@jax.jit
def _expand_tiles(w):
    return pl.pallas_call(
        _expand_body,
        out_shape=jax.ShapeDtypeStruct((_DT, _L, 8, 128), jnp.float32),
    )(w)


@functools.partial(
    pl.kernel,
    out_type=jax.ShapeDtypeStruct((_L, _DT, _BT, 8, 128), jnp.float32),
    mesh=plsc.VectorSubcoreMesh(core_axis_name="c", subcore_axis_name="s"),
    scratch_types=[
        pltpu.VMEM((_L_HALF, 8, 128), jnp.float32),
        pltpu.SemaphoreType.DMA,
    ],
)
def _sc_broadcast(x_hbm, out_hbm, xl, sem):
    c = lax.axis_index("c")
    s = lax.axis_index("s")
    l0 = c * _L_HALF
    dt = s // 2
    bt0 = (s % 2) * (_BT // 2)

    pltpu.sync_copy(x_hbm.at[dt, pl.ds(l0, _L_HALF)], xl)

    copies = [
        pltpu.async_copy(
            xl,
            out_hbm.at[pl.ds(l0, _L_HALF), dt, bt0 + k],
            sem,
        )
        for k in range(_BT // 2)
    ]
    for cp in copies:
        cp.wait()


def _fill_body(x_ref, partial_ref, o_ref):
    del partial_ref
    t = x_ref[...]
    t = jnp.transpose(t, (1, 0, 2, 3))
    o_ref[...] = jnp.broadcast_to(t[:, :, None], (_LB, _DT, _BT, 8, 128))


@jax.jit
def _tc_fill(x, partial):
    return pl.pallas_call(
        _fill_body,
        grid=(_L_TC // _LB,),
        in_specs=[
            pl.BlockSpec((_DT, _LB, 8, 128), lambda i: (0, _L_SC // _LB + i, 0, 0)),
            pl.BlockSpec(memory_space=pl.ANY),
        ],
        out_specs=pl.BlockSpec(
            (_LB, _DT, _BT, 8, 128), lambda i: (_L_SC // _LB + i, 0, 0, 0, 0)
        ),
        out_shape=jax.ShapeDtypeStruct((_L, _DT, _BT, 8, 128), jnp.float32),
        input_output_aliases={1: 0},
        compiler_params=pltpu.CompilerParams(
            dimension_semantics=("arbitrary",),
        ),
    )(x, partial)


def kernel(x, pe_weight):
    del x
    tiles = _expand_tiles(pe_weight)
    p5 = _sc_broadcast(tiles)
    p5 = _tc_fill(tiles, p5)
    return p5.transpose(2, 4, 0, 1, 3).reshape(_B, _L, _D)


# speedup vs baseline: 1.0085x; 1.0085x over previous
import functools

import jax
import jax.numpy as jnp
from jax import lax
from jax.experimental import pallas as pl
from jax.experimental.pallas import tpu as pltpu
from jax.experimental.pallas import tpu_sc as plsc

_L = 200
_D = 64
_B = 4096
_DT = _D // 8
_BT = _B // 128

_NUM_CORES = 2
_NUM_SUBCORES = 16

_L_SC = 104
_L_HALF = _L_SC // _NUM_CORES
_L_TC = _L - _L_SC
_LB = 8

def _expand_body(wt_ref, o_ref):
    dt = pl.program_id(0)
    block = wt_ref[pl.ds(dt * 8, 8), :]
    o_ref[0] = jnp.broadcast_to(block.T[:, :, None], (_L, 8, 128))


@jax.jit
def _expand_tiles(wt):
    return pl.pallas_call(
        _expand_body,
        grid=(_DT,),
        in_specs=[pl.BlockSpec((_D, _L), lambda i: (0, 0))],
        out_specs=pl.BlockSpec((1, _L, 8, 128), lambda i: (i, 0, 0, 0)),
        out_shape=jax.ShapeDtypeStruct((_DT, _L, 8, 128), jnp.float32),
        compiler_params=pltpu.CompilerParams(
            dimension_semantics=("arbitrary",),
        ),
    )(wt)


@functools.partial(
    pl.kernel,
    out_type=jax.ShapeDtypeStruct((_L, _DT, _BT, 8, 128), jnp.float32),
    mesh=plsc.VectorSubcoreMesh(core_axis_name="c", subcore_axis_name="s"),
    scratch_types=[
        pltpu.VMEM((_L_HALF, 8, 128), jnp.float32),
        pltpu.SemaphoreType.DMA,
    ],
)
def _sc_broadcast(x_hbm, out_hbm, xl, sem):
    c = lax.axis_index("c")
    s = lax.axis_index("s")
    l0 = c * _L_HALF
    dt = s // 2
    bt0 = (s % 2) * (_BT // 2)

    pltpu.sync_copy(x_hbm.at[dt, pl.ds(l0, _L_HALF)], xl)

    copies = [
        pltpu.async_copy(
            xl,
            out_hbm.at[pl.ds(l0, _L_HALF), dt, bt0 + k],
            sem,
        )
        for k in range(_BT // 2)
    ]
    for cp in copies:
        cp.wait()


def _fill_body(x_ref, partial_ref, o_ref):
    del partial_ref
    t = x_ref[...]
    t = jnp.transpose(t, (1, 0, 2, 3))
    o_ref[...] = jnp.broadcast_to(t[:, :, None], (_LB, _DT, _BT, 8, 128))


@jax.jit
def _tc_fill(x, partial):
    return pl.pallas_call(
        _fill_body,
        grid=(_L_TC // _LB,),
        in_specs=[
            pl.BlockSpec((_DT, _LB, 8, 128), lambda i: (0, _L_SC // _LB + i, 0, 0)),
            pl.BlockSpec(memory_space=pl.ANY),
        ],
        out_specs=pl.BlockSpec(
            (_LB, _DT, _BT, 8, 128), lambda i: (_L_SC // _LB + i, 0, 0, 0, 0)
        ),
        out_shape=jax.ShapeDtypeStruct((_L, _DT, _BT, 8, 128), jnp.float32),
        input_output_aliases={1: 0},
        compiler_params=pltpu.CompilerParams(
            dimension_semantics=("arbitrary",),
        ),
    )(x, partial)


def kernel(x, pe_weight):
    del x
    tiles = _expand_tiles(pe_weight.T)
    p5 = _sc_broadcast(tiles)
    p5 = _tc_fill(tiles, p5)
    return p5.transpose(2, 4, 0, 1, 3).reshape(_B, _L, _D)
